# Initial kernel scaffold; baseline (speedup 1.0000x reference)
import jax
import jax.numpy as jnp
from jax.experimental import pallas as pl


def kernel(ids, layer_num, h_skip, hps, embed_weight):
    raise NotImplementedError("write your pallas kernel here")


# speedup vs baseline: 1.0063x; 1.0063x over previous
import functools

import jax
import jax.numpy as jnp
from jax import lax
from jax.experimental import pallas as pl
from jax.experimental.pallas import tpu as pltpu
from jax.experimental.pallas import tpu_sc as plsc

N = 50000
H = 256
NC = 2
NS = 16
NW = NC * NS
B_PAD = 50176
B_PER_W = B_PAD // NW
C = 112
N_CHUNKS = B_PER_W // C


@functools.partial(
    pl.kernel,
    out_type=jax.ShapeDtypeStruct((B_PAD, H), jnp.float32),
    mesh=plsc.VectorSubcoreMesh(core_axis_name="c", subcore_axis_name="s"),
    scratch_types=[
        pltpu.VMEM((N_CHUNKS, C), jnp.int32),
        pltpu.VMEM((C, H), jnp.float32),
        pltpu.SemaphoreType.DMA,
    ],
)
def _sc_gather(table_hbm, idx_hbm, out_hbm, idx_v, rows_v, sem):
    wid = lax.axis_index("s") * NC + lax.axis_index("c")
    base = wid * B_PER_W
    pltpu.sync_copy(idx_hbm.at[wid], idx_v)

    def chunk(i, carry):
        pltpu.async_copy(table_hbm.at[idx_v.at[i]], rows_v, sem).wait()
        pltpu.sync_copy(rows_v, out_hbm.at[pl.ds(base + i * C, C)])
        return carry

    lax.fori_loop(0, N_CHUNKS, chunk, 0)


def kernel(ids, layer_num, h_skip, hps, embed_weight):
    ids_pad = jnp.concatenate(
        [ids.astype(jnp.int32), jnp.zeros((B_PAD - N,), jnp.int32)]
    ).reshape(NW, N_CHUNKS, C)
    out = _sc_gather(embed_weight, ids_pad)
    return (out[:N], h_skip)


# speedup vs baseline: 1.5166x; 1.5071x over previous
import functools

import jax
import jax.numpy as jnp
from jax import lax
from jax.experimental import pallas as pl
from jax.experimental.pallas import tpu as pltpu
from jax.experimental.pallas import tpu_sc as plsc

N = 50000
H = 256
NC = 2
NS = 16
NW = NC * NS
CH = 80
NCHUNKS = N // CH
MAXC = 20


@functools.partial(
    pl.kernel,
    out_type=jax.ShapeDtypeStruct((N, H), jnp.float32),
    mesh=plsc.VectorSubcoreMesh(core_axis_name="c", subcore_axis_name="s"),
    scratch_types=[
        pltpu.VMEM((CH,), jnp.int32),
        pltpu.VMEM((CH,), jnp.int32),
        pltpu.VMEM((CH, H), jnp.float32),
        pltpu.VMEM((CH, H), jnp.float32),
        pltpu.SemaphoreType.DMA,
        pltpu.SemaphoreType.DMA,
        pltpu.SemaphoreType.DMA,
        pltpu.SemaphoreType.DMA,
    ],
)
def _sc_gather(table_hbm, idx_hbm, out_hbm,
               idx0, idx1, rb0, rb1, sem_g0, sem_g1, sem_s0, sem_s1):
    wid = lax.axis_index("s") * NC + lax.axis_index("c")
    n_w = jnp.where(wid < 17, 20, 19)
    s_w = jnp.where(wid < 17, 20 * wid, 340 + 19 * (wid - 17))

    pltpu.sync_copy(idx_hbm.at[s_w], idx0)
    pltpu.async_copy(table_hbm.at[idx0], rb0, sem_g0)
    pltpu.sync_copy(idx_hbm.at[s_w + 1], idx1)
    pltpu.async_copy(table_hbm.at[idx1], rb1, sem_g1)

    def body(p, carry):
        a = 2 * p
        b = a + 1

        pltpu.make_async_copy(table_hbm.at[idx0], rb0, sem_g0).wait()
        pltpu.async_copy(rb0, out_hbm.at[pl.ds((s_w + a) * CH, CH)], sem_s0)

        pltpu.make_async_copy(table_hbm.at[idx1], rb1, sem_g1).wait()

        @pl.when(b < n_w)
        def _():
            pltpu.async_copy(rb1, out_hbm.at[pl.ds((s_w + b) * CH, CH)], sem_s1)

        pltpu.make_async_copy(
            rb0, out_hbm.at[pl.ds((s_w + a) * CH, CH)], sem_s0).wait()

        @pl.when(a + 2 < MAXC)
        def _():
            pltpu.sync_copy(idx_hbm.at[s_w + a + 2], idx0)
            pltpu.async_copy(table_hbm.at[idx0], rb0, sem_g0)

        @pl.when(b < n_w)
        def _():
            pltpu.make_async_copy(
                rb1, out_hbm.at[pl.ds((s_w + b) * CH, CH)], sem_s1).wait()

        @pl.when(b + 2 < MAXC)
        def _():
            pltpu.sync_copy(
                idx_hbm.at[s_w + jnp.minimum(b + 2, n_w - 1)], idx1)
            pltpu.async_copy(table_hbm.at[idx1], rb1, sem_g1)

        return carry

    lax.fori_loop(0, MAXC // 2, body, 0)


def kernel(ids, layer_num, h_skip, hps, embed_weight):
    idx = ids.astype(jnp.int32).reshape(NCHUNKS, CH)
    out = _sc_gather(embed_weight, idx)
    return (out, h_skip)


# speedup vs baseline: 1.6224x; 1.0698x over previous
import functools

import jax
import jax.numpy as jnp
from jax import lax
from jax.experimental import pallas as pl
from jax.experimental.pallas import tpu as pltpu
from jax.experimental.pallas import tpu_sc as plsc

N = 50000
H = 256
NC = 2
NS = 16
NW = NC * NS
CH = 80
NCHUNKS = N // CH
MAXC = 20


@functools.partial(
    pl.kernel,
    out_type=jax.ShapeDtypeStruct((N, H), jnp.float32),
    mesh=plsc.VectorSubcoreMesh(core_axis_name="c", subcore_axis_name="s"),
    scratch_types=[
        pltpu.VMEM((CH,), jnp.int32),
        pltpu.VMEM((CH,), jnp.int32),
        pltpu.VMEM((CH, H), jnp.float32),
        pltpu.VMEM((CH, H), jnp.float32),
        pltpu.SemaphoreType.DMA,
        pltpu.SemaphoreType.DMA,
        pltpu.SemaphoreType.DMA,
        pltpu.SemaphoreType.DMA,
    ],
)
def _sc_gather(table_hbm, idx_hbm, out_hbm,
               idx0, idx1, rb0, rb1, sem_g0, sem_g1, sem_s0, sem_s1):
    wid = lax.axis_index("s") * NC + lax.axis_index("c")
    n_w = jnp.where(wid < 17, 20, 19)
    s_w = jnp.where(wid < 17, 20 * wid, 340 + 19 * (wid - 17))

    pltpu.sync_copy(idx_hbm.at[pl.ds(s_w * CH, CH)], idx0)
    pltpu.async_copy(table_hbm.at[idx0], rb0, sem_g0)
    pltpu.sync_copy(idx_hbm.at[pl.ds((s_w + 1) * CH, CH)], idx1)
    pltpu.async_copy(table_hbm.at[idx1], rb1, sem_g1)

    def body(p, carry):
        a = 2 * p
        b = a + 1

        pltpu.make_async_copy(table_hbm.at[idx0], rb0, sem_g0).wait()
        pltpu.async_copy(rb0, out_hbm.at[pl.ds((s_w + a) * CH, CH)], sem_s0)

        pltpu.make_async_copy(table_hbm.at[idx1], rb1, sem_g1).wait()

        @pl.when(b < n_w)
        def _():
            pltpu.async_copy(rb1, out_hbm.at[pl.ds((s_w + b) * CH, CH)], sem_s1)

        pltpu.make_async_copy(
            rb0, out_hbm.at[pl.ds((s_w + a) * CH, CH)], sem_s0).wait()

        @pl.when(a + 2 < MAXC)
        def _():
            pltpu.sync_copy(idx_hbm.at[pl.ds((s_w + a + 2) * CH, CH)], idx0)
            pltpu.async_copy(table_hbm.at[idx0], rb0, sem_g0)

        @pl.when(b < n_w)
        def _():
            pltpu.make_async_copy(
                rb1, out_hbm.at[pl.ds((s_w + b) * CH, CH)], sem_s1).wait()

        @pl.when(b + 2 < MAXC)
        def _():
            pltpu.sync_copy(
                idx_hbm.at[
                    pl.ds((s_w + jnp.minimum(b + 2, n_w - 1)) * CH, CH)],
                idx1)
            pltpu.async_copy(table_hbm.at[idx1], rb1, sem_g1)

        return carry

    lax.fori_loop(0, MAXC // 2, body, 0)


_COPY_BLK = 2000


def _tc_copy_body(src_ref, dst_ref):
    dst_ref[...] = src_ref[...]


def _tc_copy(x):
    return pl.pallas_call(
        _tc_copy_body,
        out_shape=jax.ShapeDtypeStruct((N, H), jnp.float32),
        grid=(N // _COPY_BLK,),
        in_specs=[pl.BlockSpec((_COPY_BLK, H), lambda i: (i, 0))],
        out_specs=pl.BlockSpec((_COPY_BLK, H), lambda i: (i, 0)),
    )(x)


def kernel(ids, layer_num, h_skip, hps, embed_weight):
    out = _sc_gather(embed_weight, ids.astype(jnp.int32))
    return (out, _tc_copy(h_skip))
